# baseline (speedup 1.0000x reference)
import functools

import jax
import jax.numpy as jnp
from jax import lax
from jax.experimental import pallas as pl
from jax.experimental.pallas import tpu as pltpu
from jax.experimental.pallas import tpu_sc as plsc

B = 1024
K1 = 256
D = 128
NLEM = 500000
T = 0.07
MOM = 0.5

NC = 2
NS = 16
NW = NC * NS
GPT = (B * K1) // NW
CHUNK = 128
NCHUNK = GPT // CHUNK
YPT = B // NW


def _a_body(feat_ref, wt_ref, b_ref, ycol_ref, yrow_ref, fproj_ref, fe_ref):
    feat = feat_ref[...]
    proj = jnp.dot(feat, wt_ref[...], preferred_element_type=jnp.float32)
    proj = proj + b_ref[...]
    ones = jnp.ones((D, 1), jnp.float32)
    norm2 = lax.dot_general(proj * proj, ones, (((1,), (0,)), ((), ())),
                            preferred_element_type=jnp.float32)
    fproj_ref[...] = proj * lax.rsqrt(norm2)
    eq = ycol_ref[...] == yrow_ref[...]
    jidx = lax.broadcasted_iota(jnp.int32, (B, B), 1)
    li = jnp.max(jnp.where(eq, jidx, -1), axis=1, keepdims=True)
    onehot = (jidx == li).astype(jnp.float32)
    fe_ref[...] = jnp.dot(onehot, feat, preferred_element_type=jnp.float32)


def _run_a(feat, Wt, b2, ycol, yrow):
    return pl.pallas_call(
        _a_body,
        out_shape=(jax.ShapeDtypeStruct((B, D), jnp.float32),
                   jax.ShapeDtypeStruct((B, D), jnp.float32)),
    )(feat, Wt, b2, ycol, yrow)


def _b_body(idx_hbm, y_hbm, mem_hbm,
            gath_hbm, pos_hbm,
            idx_v, rows_v, y_v, yrow_v,
            gsem0, gsem1, ssem0, ssem1, sem):
    wid = lax.axis_index("s") * NC + lax.axis_index("c")
    base = pl.multiple_of(wid * GPT, 8)
    gsem = (gsem0, gsem1)
    ssem = (ssem0, ssem1)

    pltpu.sync_copy(idx_hbm.at[pl.ds(wid * YPT, YPT)], idx_v)

    def gath(c):
        p = c % 2
        isl = idx_v.at[c // 2, pl.ds((c % 2) * CHUNK, CHUNK)]
        return pltpu.make_async_copy(mem_hbm.at[isl], rows_v.at[p], gsem[p])

    def store(c):
        p = c % 2
        return pltpu.make_async_copy(
            rows_v.at[p], gath_hbm.at[pl.ds(base + c * CHUNK, CHUNK)],
            ssem[p])

    gath(0).start()
    for c in range(NCHUNK):
        if c + 1 < NCHUNK:
            if c >= 1:
                store(c - 1).wait()
            gath(c + 1).start()
        gath(c).wait()
        store(c).start()
    store(NCHUNK - 2).wait()
    store(NCHUNK - 1).wait()

    ybase = pl.multiple_of(wid * YPT, 8)
    pltpu.sync_copy(y_hbm.at[pl.ds(ybase, YPT)], y_v)
    pltpu.async_copy(mem_hbm.at[y_v], yrow_v, sem).wait()
    pltpu.sync_copy(yrow_v, pos_hbm.at[pl.ds(ybase, YPT)])


def _run_b(idx, y, memory):
    mesh = plsc.VectorSubcoreMesh(core_axis_name="c", subcore_axis_name="s")
    fn = pl.kernel(
        _b_body,
        out_type=(jax.ShapeDtypeStruct((B * K1, D), jnp.float32),
                  jax.ShapeDtypeStruct((B, D), jnp.float32)),
        mesh=mesh,
        scratch_types=[
            pltpu.VMEM((YPT, K1), jnp.int32),
            pltpu.VMEM((2, CHUNK, D), jnp.float32),
            pltpu.VMEM((YPT,), jnp.int32),
            pltpu.VMEM((YPT, D), jnp.float32),
            pltpu.SemaphoreType.DMA,
            pltpu.SemaphoreType.DMA,
            pltpu.SemaphoreType.DMA,
            pltpu.SemaphoreType.DMA,
            pltpu.SemaphoreType.DMA,
        ],
    )
    return fn(idx, y, memory)


G = 16
NG = B // G


NCSTRIPE = 8
CSTRIPE = NLEM // NCSTRIPE


def _ce_body(g_ref, f_ref, wt_ref, b_ref, mem_any, y_smem, yr_ref, fe_ref,
             out_ref, nm_any, pos_ref, sem, csem):
    i = pl.program_id(0)

    @pl.when(i == 0)
    def _():
        for j in range(NCSTRIPE):
            pltpu.make_async_copy(
                mem_any.at[pl.ds(j * CSTRIPE, CSTRIPE)],
                nm_any.at[pl.ds(j * CSTRIPE, CSTRIPE)], csem).start()
    rows = g_ref[...]
    proj = jnp.dot(rows, wt_ref[...], preferred_element_type=jnp.float32)
    proj = proj + b_ref[...]
    fblk = f_ref[...]
    num = lax.dot_general(fblk, proj, (((1,), (1,)), ((), ())),
                          preferred_element_type=jnp.float32)
    ones = jnp.ones((1, D), jnp.float32)
    norm2 = lax.dot_general(ones, proj * proj, (((1,), (1,)), ((), ())),
                            preferred_element_type=jnp.float32)
    inv = lax.rsqrt(norm2) * (1.0 / T)
    for g in range(G):
        s = slice(g * K1, (g + 1) * K1)
        out_ref[g:g + 1, :] = num[g:g + 1, s] * inv[0:1, s]

    @pl.when(i == NG - 1)
    def _():
        for j in range(NCSTRIPE):
            pltpu.make_async_copy(
                mem_any.at[pl.ds(0, CSTRIPE)],
                nm_any.at[pl.ds(0, CSTRIPE)], csem).wait()
        pos_ref[...] = yr_ref[...] * MOM + fe_ref[...] * (1.0 - MOM)

        def issue(j, _):
            r = y_smem[j]
            pltpu.make_async_copy(pos_ref.at[pl.ds(j, 1)],
                                  nm_any.at[pl.ds(r, 1)], sem).start()
            return 0

        lax.fori_loop(0, B, issue, 0)

        def drain(j, _):
            pltpu.make_async_copy(pos_ref.at[pl.ds(0, 1)],
                                  nm_any.at[pl.ds(0, 1)], sem).wait()
            return 0

        lax.fori_loop(0, B, drain, 0)


def _run_ce(gathered, fproj, Wt, b2, memory, y, yrows, feat_eff):
    out, new_memory = pl.pallas_call(
        _ce_body,
        grid=(NG,),
        in_specs=[
            pl.BlockSpec((G * K1, D), lambda i: (i, 0)),
            pl.BlockSpec((G, D), lambda i: (i, 0)),
            pl.BlockSpec((D, D), lambda i: (0, 0)),
            pl.BlockSpec((1, D), lambda i: (0, 0)),
            pl.BlockSpec(memory_space=pl.ANY),
            pl.BlockSpec(memory_space=pltpu.SMEM),
            pl.BlockSpec(memory_space=pltpu.VMEM),
            pl.BlockSpec(memory_space=pltpu.VMEM),
        ],
        out_specs=(pl.BlockSpec((G, K1), lambda i: (i, 0)),
                   pl.BlockSpec(memory_space=pl.ANY)),
        out_shape=(jax.ShapeDtypeStruct((B, K1), jnp.float32),
                   jax.ShapeDtypeStruct((NLEM, D), jnp.float32)),
        scratch_shapes=[pltpu.VMEM((B, D), jnp.float32),
                        pltpu.SemaphoreType.DMA,
                        pltpu.SemaphoreType.DMA],
    )(gathered.reshape(B * K1, D), fproj, Wt, b2, memory, y, yrows, feat_eff)
    return out, new_memory


def kernel(feat, y, idx, memory, W, b):
    Wt = W.T
    b2 = b.reshape(1, D)
    ycol = y.reshape(B, 1)
    yrow = y.reshape(1, B)
    gathered, yrows = _run_b(idx, y, memory)
    fproj, feat_eff = _run_a(feat, Wt, b2, ycol, yrow)
    out, new_memory = _run_ce(gathered, fproj, Wt, b2, memory, y,
                              yrows, feat_eff)
    return (out.reshape(B, K1, 1), new_memory)


# speedup vs baseline: 20.7372x; 20.7372x over previous
import functools

import jax
import jax.numpy as jnp
from jax import lax
from jax.experimental import pallas as pl
from jax.experimental.pallas import tpu as pltpu
from jax.experimental.pallas import tpu_sc as plsc

B = 1024
K1 = 256
D = 128
NLEM = 500000
T = 0.07
MOM = 0.5

NC = 2
NS = 16
NW = NC * NS
GPT = (B * K1) // NW
CHUNK = 128
NCHUNK = GPT // CHUNK
YPT = B // NW


def _a_body(feat_ref, wt_ref, b_ref, ycol_ref, yrow_ref, fproj_ref, fe_ref):
    feat = feat_ref[...]
    proj = jnp.dot(feat, wt_ref[...], preferred_element_type=jnp.float32)
    proj = proj + b_ref[...]
    ones = jnp.ones((D, 1), jnp.float32)
    norm2 = lax.dot_general(proj * proj, ones, (((1,), (0,)), ((), ())),
                            preferred_element_type=jnp.float32)
    fproj_ref[...] = proj * lax.rsqrt(norm2)
    eq = ycol_ref[...] == yrow_ref[...]
    jidx = lax.broadcasted_iota(jnp.int32, (B, B), 1)
    li = jnp.max(jnp.where(eq, jidx, -1), axis=1, keepdims=True)
    onehot = (jidx == li).astype(jnp.float32)
    fe_ref[...] = jnp.dot(onehot, feat, preferred_element_type=jnp.float32)


def _run_a(feat, Wt, b2, ycol, yrow):
    return pl.pallas_call(
        _a_body,
        out_shape=(jax.ShapeDtypeStruct((B, D), jnp.float32),
                   jax.ShapeDtypeStruct((B, D), jnp.float32)),
    )(feat, Wt, b2, ycol, yrow)


def _b_body(idx_hbm, y_hbm, mem_hbm,
            gath_hbm, pos_hbm,
            idx_v, rows_v, y_v, yrow_v,
            gsem0, gsem1, ssem0, ssem1, sem):
    wid = lax.axis_index("s") * NC + lax.axis_index("c")
    base = pl.multiple_of(wid * GPT, 8)
    gsem = (gsem0, gsem1)
    ssem = (ssem0, ssem1)

    pltpu.sync_copy(idx_hbm.at[pl.ds(wid * YPT, YPT)], idx_v)

    def gath(c):
        p = c % 2
        isl = idx_v.at[c // 2, pl.ds((c % 2) * CHUNK, CHUNK)]
        return pltpu.make_async_copy(mem_hbm.at[isl], rows_v.at[p], gsem[p])

    def store(c):
        p = c % 2
        return pltpu.make_async_copy(
            rows_v.at[p], gath_hbm.at[pl.ds(base + c * CHUNK, CHUNK)],
            ssem[p])

    gath(0).start()
    for c in range(NCHUNK):
        if c + 1 < NCHUNK:
            if c >= 1:
                store(c - 1).wait()
            gath(c + 1).start()
        gath(c).wait()
        store(c).start()
    store(NCHUNK - 2).wait()
    store(NCHUNK - 1).wait()

    ybase = pl.multiple_of(wid * YPT, 8)
    pltpu.sync_copy(y_hbm.at[pl.ds(ybase, YPT)], y_v)
    pltpu.async_copy(mem_hbm.at[y_v], yrow_v, sem).wait()
    pltpu.sync_copy(yrow_v, pos_hbm.at[pl.ds(ybase, YPT)])


def _run_b(idx, y, memory):
    mesh = plsc.VectorSubcoreMesh(core_axis_name="c", subcore_axis_name="s")
    fn = pl.kernel(
        _b_body,
        out_type=(jax.ShapeDtypeStruct((B * K1, D), jnp.float32),
                  jax.ShapeDtypeStruct((B, D), jnp.float32)),
        mesh=mesh,
        scratch_types=[
            pltpu.VMEM((YPT, K1), jnp.int32),
            pltpu.VMEM((2, CHUNK, D), jnp.float32),
            pltpu.VMEM((YPT,), jnp.int32),
            pltpu.VMEM((YPT, D), jnp.float32),
            pltpu.SemaphoreType.DMA,
            pltpu.SemaphoreType.DMA,
            pltpu.SemaphoreType.DMA,
            pltpu.SemaphoreType.DMA,
            pltpu.SemaphoreType.DMA,
        ],
    )
    return fn(idx, y, memory)


G = 16
NG = B // G


def _ce_body(g_ref, f_ref, wt_ref, b_ref, mem_any, y_smem, yr_ref, fe_ref,
             out_ref, nm_any, pos_ref, sem):
    del mem_any
    i = pl.program_id(0)
    rows = g_ref[...]
    proj = jnp.dot(rows.astype(jnp.bfloat16), wt_ref[...],
                   preferred_element_type=jnp.float32)
    proj = proj + b_ref[...]
    projh = proj.astype(jnp.bfloat16)
    fblk = f_ref[...]
    num = lax.dot_general(fblk, projh, (((1,), (1,)), ((), ())),
                          preferred_element_type=jnp.float32)
    ones = jnp.ones((1, D), jnp.bfloat16)
    norm2 = lax.dot_general(ones, projh * projh, (((1,), (1,)), ((), ())),
                            preferred_element_type=jnp.float32)
    inv = lax.rsqrt(norm2) * (1.0 / T)
    for g in range(G):
        s = slice(g * K1, (g + 1) * K1)
        out_ref[g:g + 1, :] = num[g:g + 1, s] * inv[0:1, s]

    @pl.when(i == NG - 1)
    def _():
        pos_ref[...] = yr_ref[...] * MOM + fe_ref[...] * (1.0 - MOM)

        def issue(j, _):
            r = y_smem[j]
            pltpu.make_async_copy(pos_ref.at[pl.ds(j, 1)],
                                  nm_any.at[pl.ds(r, 1)], sem).start()
            return 0

        lax.fori_loop(0, B, issue, 0)

        def drain(j, _):
            pltpu.make_async_copy(pos_ref.at[pl.ds(0, 1)],
                                  nm_any.at[pl.ds(0, 1)], sem).wait()
            return 0

        lax.fori_loop(0, B, drain, 0)


def _run_ce(gathered, fproj, Wt, b2, memory, y, yrows, feat_eff):
    out, new_memory = pl.pallas_call(
        _ce_body,
        grid=(NG,),
        in_specs=[
            pl.BlockSpec((G * K1, D), lambda i: (i, 0)),
            pl.BlockSpec((G, D), lambda i: (i, 0)),
            pl.BlockSpec((D, D), lambda i: (0, 0)),
            pl.BlockSpec((1, D), lambda i: (0, 0)),
            pl.BlockSpec(memory_space=pl.ANY),
            pl.BlockSpec(memory_space=pltpu.SMEM),
            pl.BlockSpec(memory_space=pltpu.VMEM),
            pl.BlockSpec(memory_space=pltpu.VMEM),
        ],
        out_specs=(pl.BlockSpec((G, K1), lambda i: (i, 0)),
                   pl.BlockSpec(memory_space=pl.ANY)),
        out_shape=(jax.ShapeDtypeStruct((B, K1), jnp.float32),
                   jax.ShapeDtypeStruct((NLEM, D), jnp.float32)),
        input_output_aliases={4: 1},
        scratch_shapes=[pltpu.VMEM((B, D), jnp.float32),
                        pltpu.SemaphoreType.DMA],
    )(gathered.reshape(B * K1, D), fproj.astype(jnp.bfloat16),
      Wt.astype(jnp.bfloat16), b2, memory, y, yrows, feat_eff)
    return out, new_memory


def kernel(feat, y, idx, memory, W, b):
    Wt = W.T
    b2 = b.reshape(1, D)
    ycol = y.reshape(B, 1)
    yrow = y.reshape(1, B)
    gathered, yrows = _run_b(idx, y, memory)
    fproj, feat_eff = _run_a(feat, Wt, b2, ycol, yrow)
    out, new_memory = _run_ce(gathered, fproj, Wt, b2, memory, y,
                              yrows, feat_eff)
    return (out.reshape(B, K1, 1), new_memory)


# speedup vs baseline: 21.9239x; 1.0572x over previous
import functools

import jax
import jax.numpy as jnp
from jax import lax
from jax.experimental import pallas as pl
from jax.experimental.pallas import tpu as pltpu
from jax.experimental.pallas import tpu_sc as plsc

B = 1024
K1 = 256
D = 128
NLEM = 500000
T = 0.07
MOM = 0.5

NC = 2
NS = 16
NW = NC * NS
GPT = (B * K1) // NW
CHUNK = 128
NCHUNK = GPT // CHUNK
YPT = B // NW


def _a_body(feat_ref, wt_ref, b_ref, ycol_ref, yrow_ref, fproj_ref, fe_ref):
    feat = feat_ref[...]
    proj = jnp.dot(feat, wt_ref[...], preferred_element_type=jnp.float32)
    proj = proj + b_ref[...]
    ones = jnp.ones((D, 1), jnp.float32)
    norm2 = lax.dot_general(proj * proj, ones, (((1,), (0,)), ((), ())),
                            preferred_element_type=jnp.float32)
    fproj_ref[...] = proj * lax.rsqrt(norm2)
    eq = ycol_ref[...] == yrow_ref[...]
    jidx = lax.broadcasted_iota(jnp.int32, (B, B), 1)
    li = jnp.max(jnp.where(eq, jidx, -1), axis=1, keepdims=True)
    onehot = (jidx == li).astype(jnp.float32)
    fe_ref[...] = jnp.dot(onehot, feat, preferred_element_type=jnp.float32)


def _run_a(feat, Wt, b2, ycol, yrow):
    return pl.pallas_call(
        _a_body,
        out_shape=(jax.ShapeDtypeStruct((B, D), jnp.float32),
                   jax.ShapeDtypeStruct((B, D), jnp.float32)),
    )(feat, Wt, b2, ycol, yrow)


def _b_body(idx_hbm, y_hbm, mem_hbm,
            gath_hbm, pos_hbm,
            idx_v, rows_v, y_v, yrow_v,
            gsem0, gsem1, ssem0, ssem1, sem):
    wid = lax.axis_index("s") * NC + lax.axis_index("c")
    base = pl.multiple_of(wid * GPT, 8)
    gsem = (gsem0, gsem1)
    ssem = (ssem0, ssem1)

    pltpu.sync_copy(idx_hbm.at[pl.ds(wid * YPT, YPT)], idx_v)

    def gath(c):
        p = c % 2
        isl = idx_v.at[c // 2, pl.ds((c % 2) * CHUNK, CHUNK)]
        return pltpu.make_async_copy(mem_hbm.at[isl], rows_v.at[p], gsem[p])

    def store(c):
        p = c % 2
        return pltpu.make_async_copy(
            rows_v.at[p], gath_hbm.at[pl.ds(base + c * CHUNK, CHUNK)],
            ssem[p])

    gath(0).start()
    for c in range(NCHUNK):
        if c + 1 < NCHUNK:
            if c >= 1:
                store(c - 1).wait()
            gath(c + 1).start()
        gath(c).wait()
        store(c).start()
    store(NCHUNK - 2).wait()
    store(NCHUNK - 1).wait()

    ybase = pl.multiple_of(wid * YPT, 8)
    pltpu.sync_copy(y_hbm.at[pl.ds(ybase, YPT)], y_v)
    pltpu.async_copy(mem_hbm.at[y_v], yrow_v, sem).wait()
    pltpu.sync_copy(yrow_v, pos_hbm.at[pl.ds(ybase, YPT)])


def _run_b(idx, y, memory):
    mesh = plsc.VectorSubcoreMesh(core_axis_name="c", subcore_axis_name="s")
    fn = pl.kernel(
        _b_body,
        out_type=(jax.ShapeDtypeStruct((B * K1, D), jnp.float32),
                  jax.ShapeDtypeStruct((B, D), jnp.float32)),
        mesh=mesh,
        scratch_types=[
            pltpu.VMEM((YPT, K1), jnp.int32),
            pltpu.VMEM((2, CHUNK, D), jnp.float32),
            pltpu.VMEM((YPT,), jnp.int32),
            pltpu.VMEM((YPT, D), jnp.float32),
            pltpu.SemaphoreType.DMA,
            pltpu.SemaphoreType.DMA,
            pltpu.SemaphoreType.DMA,
            pltpu.SemaphoreType.DMA,
            pltpu.SemaphoreType.DMA,
        ],
    )
    return fn(idx, y, memory)


G = 32
NG = B // G


def _ce_body(g_ref, f_ref, wt_ref, b_ref, mem_any, y_smem, yr_ref, fe_ref,
             out_ref, nm_any, pos_ref, sem):
    del mem_any
    i = pl.program_id(0)
    rows = g_ref[...]
    proj = jnp.dot(rows.astype(jnp.bfloat16), wt_ref[...],
                   preferred_element_type=jnp.float32)
    proj = proj + b_ref[...]
    projh = proj.astype(jnp.bfloat16)
    fblk = f_ref[...]
    num = lax.dot_general(fblk, projh, (((1,), (1,)), ((), ())),
                          preferred_element_type=jnp.float32)
    ones = jnp.ones((1, D), jnp.bfloat16)
    norm2 = lax.dot_general(ones, projh * projh, (((1,), (1,)), ((), ())),
                            preferred_element_type=jnp.float32)
    inv = lax.rsqrt(norm2) * (1.0 / T)
    for g in range(G):
        s = slice(g * K1, (g + 1) * K1)
        out_ref[g:g + 1, :] = num[g:g + 1, s] * inv[0:1, s]

    @pl.when(i == NG - 1)
    def _():
        pos_ref[...] = yr_ref[...] * MOM + fe_ref[...] * (1.0 - MOM)

        def issue(j, _):
            r = y_smem[j]
            pltpu.make_async_copy(pos_ref.at[pl.ds(j, 1)],
                                  nm_any.at[pl.ds(r, 1)], sem).start()
            return 0

        lax.fori_loop(0, B, issue, 0)

        def drain(j, _):
            pltpu.make_async_copy(pos_ref.at[pl.ds(0, 1)],
                                  nm_any.at[pl.ds(0, 1)], sem).wait()
            return 0

        lax.fori_loop(0, B, drain, 0)


def _run_ce(gathered, fproj, Wt, b2, memory, y, yrows, feat_eff):
    out, new_memory = pl.pallas_call(
        _ce_body,
        grid=(NG,),
        in_specs=[
            pl.BlockSpec((G * K1, D), lambda i: (i, 0)),
            pl.BlockSpec((G, D), lambda i: (i, 0)),
            pl.BlockSpec((D, D), lambda i: (0, 0)),
            pl.BlockSpec((1, D), lambda i: (0, 0)),
            pl.BlockSpec(memory_space=pl.ANY),
            pl.BlockSpec(memory_space=pltpu.SMEM),
            pl.BlockSpec(memory_space=pltpu.VMEM),
            pl.BlockSpec(memory_space=pltpu.VMEM),
        ],
        out_specs=(pl.BlockSpec((G, K1), lambda i: (i, 0)),
                   pl.BlockSpec(memory_space=pl.ANY)),
        out_shape=(jax.ShapeDtypeStruct((B, K1), jnp.float32),
                   jax.ShapeDtypeStruct((NLEM, D), jnp.float32)),
        input_output_aliases={4: 1},
        scratch_shapes=[pltpu.VMEM((B, D), jnp.float32),
                        pltpu.SemaphoreType.DMA],
    )(gathered.reshape(B * K1, D), fproj.astype(jnp.bfloat16),
      Wt.astype(jnp.bfloat16), b2, memory, y, yrows, feat_eff)
    return out, new_memory


def kernel(feat, y, idx, memory, W, b):
    Wt = W.T
    b2 = b.reshape(1, D)
    ycol = y.reshape(B, 1)
    yrow = y.reshape(1, B)
    gathered, yrows = _run_b(idx, y, memory)
    fproj, feat_eff = _run_a(feat, Wt, b2, ycol, yrow)
    out, new_memory = _run_ce(gathered, fproj, Wt, b2, memory, y,
                              yrows, feat_eff)
    return (out.reshape(B, K1, 1), new_memory)


# speedup vs baseline: 22.0510x; 1.0058x over previous
import functools

import jax
import jax.numpy as jnp
from jax import lax
from jax.experimental import pallas as pl
from jax.experimental.pallas import tpu as pltpu
from jax.experimental.pallas import tpu_sc as plsc

B = 1024
K1 = 256
D = 128
NLEM = 500000
T = 0.07
MOM = 0.5

NC = 2
NS = 16
NW = NC * NS
GPT = (B * K1) // NW
CHUNK = 128
NCHUNK = GPT // CHUNK
YPT = B // NW


def _a_body(feat_ref, wt_ref, b_ref, ycol_ref, yrow_ref, fproj_ref, fe_ref):
    feat = feat_ref[...]
    proj = jnp.dot(feat, wt_ref[...], preferred_element_type=jnp.float32)
    proj = proj + b_ref[...]
    ones = jnp.ones((D, 1), jnp.float32)
    norm2 = lax.dot_general(proj * proj, ones, (((1,), (0,)), ((), ())),
                            preferred_element_type=jnp.float32)
    fproj_ref[...] = proj * lax.rsqrt(norm2)
    eq = ycol_ref[...] == yrow_ref[...]
    jidx = lax.broadcasted_iota(jnp.int32, (B, B), 1)
    li = jnp.max(jnp.where(eq, jidx, -1), axis=1, keepdims=True)
    onehot = (jidx == li).astype(jnp.float32)
    fe_ref[...] = jnp.dot(onehot, feat, preferred_element_type=jnp.float32)


def _run_a(feat, Wt, b2, ycol, yrow):
    return pl.pallas_call(
        _a_body,
        out_shape=(jax.ShapeDtypeStruct((B, D), jnp.float32),
                   jax.ShapeDtypeStruct((B, D), jnp.float32)),
    )(feat, Wt, b2, ycol, yrow)


def _b_body(idx_hbm, y_hbm, mem_hbm,
            gath_hbm, pos_hbm,
            idx_v, rows_v, y_v, yrow_v,
            gsem0, gsem1, ssem0, ssem1, sem):
    wid = lax.axis_index("s") * NC + lax.axis_index("c")
    base = pl.multiple_of(wid * GPT, 8)
    gsem = (gsem0, gsem1)
    ssem = (ssem0, ssem1)

    pltpu.sync_copy(idx_hbm.at[pl.ds(wid * YPT, YPT)], idx_v)

    def gath(c):
        p = c % 2
        isl = idx_v.at[c // 2, pl.ds((c % 2) * CHUNK, CHUNK)]
        return pltpu.make_async_copy(mem_hbm.at[isl], rows_v.at[p], gsem[p])

    def store(c):
        p = c % 2
        return pltpu.make_async_copy(
            rows_v.at[p], gath_hbm.at[pl.ds(base + c * CHUNK, CHUNK)],
            ssem[p])

    gath(0).start()
    for c in range(NCHUNK):
        if c + 1 < NCHUNK:
            if c >= 1:
                store(c - 1).wait()
            gath(c + 1).start()
        gath(c).wait()
        store(c).start()
    store(NCHUNK - 2).wait()
    store(NCHUNK - 1).wait()

    ybase = pl.multiple_of(wid * YPT, 8)
    pltpu.sync_copy(y_hbm.at[pl.ds(ybase, YPT)], y_v)
    pltpu.async_copy(mem_hbm.at[y_v], yrow_v, sem).wait()
    pltpu.sync_copy(yrow_v, pos_hbm.at[pl.ds(ybase, YPT)])


def _run_b(idx, y, memory):
    mesh = plsc.VectorSubcoreMesh(core_axis_name="c", subcore_axis_name="s")
    fn = pl.kernel(
        _b_body,
        out_type=(jax.ShapeDtypeStruct((B * K1, D), jnp.float32),
                  jax.ShapeDtypeStruct((B, D), jnp.float32)),
        mesh=mesh,
        scratch_types=[
            pltpu.VMEM((YPT, K1), jnp.int32),
            pltpu.VMEM((2, CHUNK, D), jnp.float32),
            pltpu.VMEM((YPT,), jnp.int32),
            pltpu.VMEM((YPT, D), jnp.float32),
            pltpu.SemaphoreType.DMA,
            pltpu.SemaphoreType.DMA,
            pltpu.SemaphoreType.DMA,
            pltpu.SemaphoreType.DMA,
            pltpu.SemaphoreType.DMA,
        ],
    )
    return fn(idx, y, memory)


G = 64
NG = B // G


def _ce_body(g_ref, f_ref, wt_ref, b_ref, mem_any, y_smem, yr_ref, fe_ref,
             out_ref, nm_any, pos_ref, sem):
    del mem_any
    i = pl.program_id(0)
    rows = g_ref[...]
    proj = jnp.dot(rows.astype(jnp.bfloat16), wt_ref[...],
                   preferred_element_type=jnp.float32)
    proj = proj + b_ref[...]
    projh = proj.astype(jnp.bfloat16)
    fblk = f_ref[...]
    num = lax.dot_general(fblk, projh, (((1,), (1,)), ((), ())),
                          preferred_element_type=jnp.float32)
    ones = jnp.ones((1, D), jnp.bfloat16)
    norm2 = lax.dot_general(ones, projh * projh, (((1,), (1,)), ((), ())),
                            preferred_element_type=jnp.float32)
    inv = lax.rsqrt(norm2) * (1.0 / T)
    for g in range(G):
        s = slice(g * K1, (g + 1) * K1)
        out_ref[g:g + 1, :] = num[g:g + 1, s] * inv[0:1, s]

    @pl.when(i == NG - 1)
    def _():
        pos_ref[...] = yr_ref[...] * MOM + fe_ref[...] * (1.0 - MOM)

        def issue(j, _):
            r = y_smem[j]
            pltpu.make_async_copy(pos_ref.at[pl.ds(j, 1)],
                                  nm_any.at[pl.ds(r, 1)], sem).start()
            return 0

        lax.fori_loop(0, B, issue, 0)

        def drain(j, _):
            pltpu.make_async_copy(pos_ref.at[pl.ds(0, 1)],
                                  nm_any.at[pl.ds(0, 1)], sem).wait()
            return 0

        lax.fori_loop(0, B, drain, 0)


def _run_ce(gathered, fproj, Wt, b2, memory, y, yrows, feat_eff):
    out, new_memory = pl.pallas_call(
        _ce_body,
        grid=(NG,),
        in_specs=[
            pl.BlockSpec((G * K1, D), lambda i: (i, 0)),
            pl.BlockSpec((G, D), lambda i: (i, 0)),
            pl.BlockSpec((D, D), lambda i: (0, 0)),
            pl.BlockSpec((1, D), lambda i: (0, 0)),
            pl.BlockSpec(memory_space=pl.ANY),
            pl.BlockSpec(memory_space=pltpu.SMEM),
            pl.BlockSpec(memory_space=pltpu.VMEM),
            pl.BlockSpec(memory_space=pltpu.VMEM),
        ],
        out_specs=(pl.BlockSpec((G, K1), lambda i: (i, 0)),
                   pl.BlockSpec(memory_space=pl.ANY)),
        out_shape=(jax.ShapeDtypeStruct((B, K1), jnp.float32),
                   jax.ShapeDtypeStruct((NLEM, D), jnp.float32)),
        input_output_aliases={4: 1},
        scratch_shapes=[pltpu.VMEM((B, D), jnp.float32),
                        pltpu.SemaphoreType.DMA],
    )(gathered.reshape(B * K1, D), fproj.astype(jnp.bfloat16),
      Wt.astype(jnp.bfloat16), b2, memory, y, yrows, feat_eff)
    return out, new_memory


def kernel(feat, y, idx, memory, W, b):
    Wt = W.T
    b2 = b.reshape(1, D)
    ycol = y.reshape(B, 1)
    yrow = y.reshape(1, B)
    gathered, yrows = _run_b(idx, y, memory)
    fproj, feat_eff = _run_a(feat, Wt, b2, ycol, yrow)
    out, new_memory = _run_ce(gathered, fproj, Wt, b2, memory, y,
                              yrows, feat_eff)
    return (out.reshape(B, K1, 1), new_memory)


# speedup vs baseline: 22.2056x; 1.0070x over previous
import functools

import jax
import jax.numpy as jnp
from jax import lax
from jax.experimental import pallas as pl
from jax.experimental.pallas import tpu as pltpu
from jax.experimental.pallas import tpu_sc as plsc

B = 1024
K1 = 256
D = 128
NLEM = 500000
T = 0.07
MOM = 0.5

NC = 2
NS = 16
NW = NC * NS
GPT = (B * K1) // NW
CHUNK = 128
NCHUNK = GPT // CHUNK
YPT = B // NW


def _a_body(feat_ref, wt_ref, b_ref, ycol_ref, yrow_ref, fproj_ref, fe_ref):
    feat = feat_ref[...]
    proj = jnp.dot(feat, wt_ref[...], preferred_element_type=jnp.float32)
    proj = proj + b_ref[...]
    ones = jnp.ones((D, 1), jnp.float32)
    norm2 = lax.dot_general(proj * proj, ones, (((1,), (0,)), ((), ())),
                            preferred_element_type=jnp.float32)
    fproj_ref[...] = proj * lax.rsqrt(norm2)
    eq = ycol_ref[...] == yrow_ref[...]
    jidx = lax.broadcasted_iota(jnp.int32, (B, B), 1)
    li = jnp.max(jnp.where(eq, jidx, -1), axis=1, keepdims=True)
    onehot = (jidx == li).astype(jnp.float32)
    fe_ref[...] = jnp.dot(onehot, feat, preferred_element_type=jnp.float32)


def _run_a(feat, Wt, b2, ycol, yrow):
    return pl.pallas_call(
        _a_body,
        out_shape=(jax.ShapeDtypeStruct((B, D), jnp.float32),
                   jax.ShapeDtypeStruct((B, D), jnp.float32)),
    )(feat, Wt, b2, ycol, yrow)


NBUF = 4


def _b_body(idx_hbm, y_hbm, mem_hbm,
            gath_hbm, pos_hbm,
            idx_v, rows_v, y_v, yrow_v,
            gsem0, gsem1, gsem2, gsem3, ssem0, ssem1, ssem2, ssem3, sem):
    wid = lax.axis_index("s") * NC + lax.axis_index("c")
    base = pl.multiple_of(wid * GPT, 8)
    gsem = (gsem0, gsem1, gsem2, gsem3)
    ssem = (ssem0, ssem1, ssem2, ssem3)

    pltpu.sync_copy(idx_hbm.at[pl.ds(wid * YPT, YPT)], idx_v)

    def gath(c):
        p = c % NBUF
        isl = idx_v.at[c // 2, pl.ds((c % 2) * CHUNK, CHUNK)]
        return pltpu.make_async_copy(mem_hbm.at[isl], rows_v.at[p], gsem[p])

    def store(c):
        p = c % NBUF
        return pltpu.make_async_copy(
            rows_v.at[p], gath_hbm.at[pl.ds(base + c * CHUNK, CHUNK)],
            ssem[p])

    gath(0).start()
    gath(1).start()
    for c in range(NCHUNK):
        if c + 2 < NCHUNK:
            if c >= 2:
                store(c - 2).wait()
            gath(c + 2).start()
        gath(c).wait()
        store(c).start()
    for c in range(NCHUNK - 4, NCHUNK):
        store(c).wait()

    ybase = pl.multiple_of(wid * YPT, 8)
    pltpu.sync_copy(y_hbm.at[pl.ds(ybase, YPT)], y_v)
    pltpu.async_copy(mem_hbm.at[y_v], yrow_v, sem).wait()
    pltpu.sync_copy(yrow_v, pos_hbm.at[pl.ds(ybase, YPT)])


def _run_b(idx, y, memory):
    mesh = plsc.VectorSubcoreMesh(core_axis_name="c", subcore_axis_name="s")
    fn = pl.kernel(
        _b_body,
        out_type=(jax.ShapeDtypeStruct((B * K1, D), jnp.float32),
                  jax.ShapeDtypeStruct((B, D), jnp.float32)),
        mesh=mesh,
        scratch_types=[
            pltpu.VMEM((YPT, K1), jnp.int32),
            pltpu.VMEM((NBUF, CHUNK, D), jnp.float32),
            pltpu.VMEM((YPT,), jnp.int32),
            pltpu.VMEM((YPT, D), jnp.float32),
            pltpu.SemaphoreType.DMA,
            pltpu.SemaphoreType.DMA,
            pltpu.SemaphoreType.DMA,
            pltpu.SemaphoreType.DMA,
            pltpu.SemaphoreType.DMA,
            pltpu.SemaphoreType.DMA,
            pltpu.SemaphoreType.DMA,
            pltpu.SemaphoreType.DMA,
            pltpu.SemaphoreType.DMA,
        ],
    )
    return fn(idx, y, memory)


G = 64
NG = B // G


def _ce_body(g_ref, f_ref, wt_ref, b_ref, mem_any, y_smem, yr_ref, fe_ref,
             out_ref, nm_any, pos_ref, sem):
    del mem_any
    i = pl.program_id(0)
    rows = g_ref[...]
    proj = jnp.dot(rows.astype(jnp.bfloat16), wt_ref[...],
                   preferred_element_type=jnp.float32)
    proj = proj + b_ref[...]
    projh = proj.astype(jnp.bfloat16)
    fblk = f_ref[...]
    num = lax.dot_general(fblk, projh, (((1,), (1,)), ((), ())),
                          preferred_element_type=jnp.float32)
    ones = jnp.ones((1, D), jnp.bfloat16)
    norm2 = lax.dot_general(ones, projh * projh, (((1,), (1,)), ((), ())),
                            preferred_element_type=jnp.float32)
    inv = lax.rsqrt(norm2) * (1.0 / T)
    for g in range(G):
        s = slice(g * K1, (g + 1) * K1)
        out_ref[g:g + 1, :] = num[g:g + 1, s] * inv[0:1, s]

    @pl.when(i == NG - 1)
    def _():
        pos_ref[...] = yr_ref[...] * MOM + fe_ref[...] * (1.0 - MOM)

        def issue(j, _):
            r = y_smem[j]
            pltpu.make_async_copy(pos_ref.at[pl.ds(j, 1)],
                                  nm_any.at[pl.ds(r, 1)], sem).start()
            return 0

        lax.fori_loop(0, B, issue, 0)

        def drain(j, _):
            pltpu.make_async_copy(pos_ref.at[pl.ds(0, 1)],
                                  nm_any.at[pl.ds(0, 1)], sem).wait()
            return 0

        lax.fori_loop(0, B, drain, 0)


def _run_ce(gathered, fproj, Wt, b2, memory, y, yrows, feat_eff):
    out, new_memory = pl.pallas_call(
        _ce_body,
        grid=(NG,),
        in_specs=[
            pl.BlockSpec((G * K1, D), lambda i: (i, 0)),
            pl.BlockSpec((G, D), lambda i: (i, 0)),
            pl.BlockSpec((D, D), lambda i: (0, 0)),
            pl.BlockSpec((1, D), lambda i: (0, 0)),
            pl.BlockSpec(memory_space=pl.ANY),
            pl.BlockSpec(memory_space=pltpu.SMEM),
            pl.BlockSpec(memory_space=pltpu.VMEM),
            pl.BlockSpec(memory_space=pltpu.VMEM),
        ],
        out_specs=(pl.BlockSpec((G, K1), lambda i: (i, 0)),
                   pl.BlockSpec(memory_space=pl.ANY)),
        out_shape=(jax.ShapeDtypeStruct((B, K1), jnp.float32),
                   jax.ShapeDtypeStruct((NLEM, D), jnp.float32)),
        input_output_aliases={4: 1},
        scratch_shapes=[pltpu.VMEM((B, D), jnp.float32),
                        pltpu.SemaphoreType.DMA],
    )(gathered.reshape(B * K1, D), fproj.astype(jnp.bfloat16),
      Wt.astype(jnp.bfloat16), b2, memory, y, yrows, feat_eff)
    return out, new_memory


def kernel(feat, y, idx, memory, W, b):
    Wt = W.T
    b2 = b.reshape(1, D)
    ycol = y.reshape(B, 1)
    yrow = y.reshape(1, B)
    gathered, yrows = _run_b(idx, y, memory)
    fproj, feat_eff = _run_a(feat, Wt, b2, ycol, yrow)
    out, new_memory = _run_ce(gathered, fproj, Wt, b2, memory, y,
                              yrows, feat_eff)
    return (out.reshape(B, K1, 1), new_memory)
